# Initial kernel scaffold; baseline (speedup 1.0000x reference)
import jax
import jax.numpy as jnp
from jax.experimental import pallas as pl


def kernel(x_question, x_answer, pos_edge_label_index, neg_edge_label_index):
    raise NotImplementedError("write your pallas kernel here")


# speedup vs baseline: 8.7691x; 8.7691x over previous
import functools

import jax
import jax.numpy as jnp
from jax import lax
from jax.experimental import pallas as pl
from jax.experimental.pallas import tpu as pltpu
from jax.experimental.pallas import tpu_sc as plsc

_N_NODES = 10000
_D = 128
_E = 320000
_ETOT = 2 * _E
_NC = 2
_NS = 16
_NW = _NC * _NS
_EW = _ETOT // _NW
_B = 80
_NCHUNK = _EW // _B
_LANES = 16
_NVREG = _D // _LANES
_STRIDE = _LANES + 1


def _dot_kernel(xq_hbm, xa_hbm, idxq_hbm, idxa_hbm, out_hbm,
                idxq_v, idxa_v, out_v, rowsq_v, rowsa_v, part_v, sem0, sem1):
    sems = (sem0, sem1)
    wid = lax.axis_index("s") * _NC + lax.axis_index("c")
    base = wid * _EW

    pltpu.sync_copy(idxq_hbm.at[pl.ds(base, _EW)], idxq_v)
    pltpu.sync_copy(idxa_hbm.at[pl.ds(base, _EW)], idxa_v)

    def copies(c, s):
        iq = idxq_v.at[pl.ds(c * _B, _B)]
        ia = idxa_v.at[pl.ds(c * _B, _B)]
        return (pltpu.make_async_copy(xq_hbm.at[iq], rowsq_v.at[s], sems[s]),
                pltpu.make_async_copy(xa_hbm.at[ia], rowsa_v.at[s], sems[s]))

    def issue(c, s):
        for cp in copies(c, s):
            cp.start()

    issue(0, 0)
    issue(1, 1)

    col_base = lax.iota(jnp.int32, _LANES) * _STRIDE

    @pl.loop(0, _NCHUNK, step=2)
    def _chunk_pair(cpair):
        for s in range(2):
            c = cpair + s
            for cp in copies(c, s):
                cp.wait()
            off = c * _B

            @pl.loop(0, _B // _LANES)
            def _group(g):
                e0 = g * _LANES
                for j in range(_LANES):
                    e = e0 + j
                    acc = (rowsq_v[s, e, pl.ds(0, _LANES)] *
                           rowsa_v[s, e, pl.ds(0, _LANES)])
                    for k in range(1, _NVREG):
                        acc = acc + (rowsq_v[s, e, pl.ds(k * _LANES, _LANES)] *
                                     rowsa_v[s, e, pl.ds(k * _LANES, _LANES)])
                    part_v[pl.ds(j * _STRIDE, _LANES)] = acc
                res = plsc.load_gather(part_v, [col_base])
                for l in range(1, _LANES):
                    res = res + plsc.load_gather(part_v, [col_base + l])
                out_v[pl.ds(off + e0, _LANES)] = res

            @pl.when(c + 2 < _NCHUNK)
            def _():
                issue(c + 2, s)

    pltpu.sync_copy(out_v, out_hbm.at[pl.ds(base, _EW)])


@jax.jit
def _run(xq, xa, idxq, idxa):
    mesh = plsc.VectorSubcoreMesh(core_axis_name="c", subcore_axis_name="s")
    fn = pl.kernel(
        _dot_kernel,
        out_type=jax.ShapeDtypeStruct((_ETOT,), jnp.float32),
        mesh=mesh,
        compiler_params=pltpu.CompilerParams(needs_layout_passes=False),
        scratch_types=[
            pltpu.VMEM((_EW,), jnp.int32),
            pltpu.VMEM((_EW,), jnp.int32),
            pltpu.VMEM((_EW,), jnp.float32),
            pltpu.VMEM((2, _B, _D), jnp.float32),
            pltpu.VMEM((2, _B, _D), jnp.float32),
            pltpu.VMEM((_LANES * _STRIDE,), jnp.float32),
            pltpu.SemaphoreType.DMA,
            pltpu.SemaphoreType.DMA,
        ],
    )
    return fn(xq, xa, idxq, idxa)


def kernel(x_question, x_answer, pos_edge_label_index, neg_edge_label_index):
    idx = jnp.concatenate([pos_edge_label_index, neg_edge_label_index], axis=1)
    out = _run(x_question, x_answer, idx[0], idx[1])
    return out[:_E], out[_E:]
